# baseline (speedup 1.0000x reference)
import functools

import jax
import jax.numpy as jnp
from jax import lax
from jax.experimental import pallas as pl
from jax.experimental.pallas import tpu as pltpu
from jax.experimental.pallas import tpu_sc as plsc

_INV_SQRT2 = 0.7071067811865476

_NT = 16384
_H = 4096

_NT_TC = 8960
_NT_SC = _NT - _NT_TC

_BLOCK_T = 256


def _tc_body(moe_ref, mlp_ref, out_ref):
    out_ref[...] = (moe_ref[:, 0, :] + moe_ref[:, 1, :] + mlp_ref[...]) * _INV_SQRT2


def _tc_combine(moe, mlp):
    return pl.pallas_call(
        _tc_body,
        grid=(_NT_TC // _BLOCK_T,),
        in_specs=[
            pl.BlockSpec((_BLOCK_T, 2, _H), lambda i: (i, 0, 0)),
            pl.BlockSpec((_BLOCK_T, _H), lambda i: (i, 0)),
        ],
        out_specs=pl.BlockSpec((_BLOCK_T, _H), lambda i: (i, 0)),
        out_shape=jax.ShapeDtypeStruct((_NT, _H), jnp.float32),
    )(moe, mlp)


_LANES = 16
_NC = 2
_NS = 16
_NW = _NC * _NS
_TOK_PER_W = _NT_SC // _NW
_T = 2
_NCHUNK = _TOK_PER_W // _T
_NBI = 4
_NBO = 2


def _sc_body(moe_hbm, mlp_hbm, out_hbm, moe_v, mlp_v, out_v,
             sem_moe, sem_mlp, sem_out):
    wid = lax.axis_index("s") * _NC + lax.axis_index("c")
    base_in = _NT_TC + wid * _TOK_PER_W
    base_out = wid * _TOK_PER_W

    def in_moe(ci, b):
        tok = base_in + ci * _T
        return pltpu.make_async_copy(
            moe_hbm.at[pl.ds(tok, _T)], moe_v.at[b], sem_moe.at[b])

    def in_mlp(ci, b):
        tok = base_in + ci * _T
        return pltpu.make_async_copy(
            mlp_hbm.at[pl.ds(tok, _T)], mlp_v.at[b], sem_mlp.at[b])

    def out_cp(ci, b):
        tok = base_out + ci * _T
        return pltpu.make_async_copy(
            out_v.at[b], out_hbm.at[pl.ds(tok, _T)], sem_out.at[b])

    for b in range(_NBI):
        in_moe(b, b).start()
        in_mlp(b, b).start()

    @pl.loop(0, _NCHUNK, step=_NBI)
    def _(ci):
        for b in range(_NBI):
            cj = ci + b
            ob = b % _NBO
            in_moe(cj, b).wait()
            in_mlp(cj, b).wait()

            @pl.when(cj >= _NBO)
            def _(cj=cj, ob=ob):
                out_cp(cj - _NBO, ob).wait()

            for t in range(_T):
                @plsc.parallel_loop(0, _H // _LANES, unroll=8)
                def _(j, t=t, b=b, ob=ob):
                    sl = pl.ds(j * _LANES, _LANES)
                    out_v[ob, t, sl] = (
                        moe_v[b, t, 0, sl] + moe_v[b, t, 1, sl]
                        + mlp_v[b, t, sl]) * _INV_SQRT2

            out_cp(cj, ob).start()

            @pl.when(cj + _NBI < _NCHUNK)
            def _(cj=cj, b=b):
                in_moe(cj + _NBI, b).start()
                in_mlp(cj + _NBI, b).start()

    for b in range(_NBO):
        out_cp(_NCHUNK - _NBO + b, b).wait()


_sc_combine = functools.partial(
    pl.kernel,
    out_type=jax.ShapeDtypeStruct((_NT_SC, _H), jnp.float32),
    mesh=plsc.VectorSubcoreMesh(
        core_axis_name="c", subcore_axis_name="s",
        num_cores=_NC, num_subcores=_NS),
    scratch_types=[
        pltpu.VMEM((_NBI, _T, 2, _H), jnp.float32),
        pltpu.VMEM((_NBI, _T, _H), jnp.float32),
        pltpu.VMEM((_NBO, _T, _H), jnp.float32),
        pltpu.SemaphoreType.DMA((_NBI,)),
        pltpu.SemaphoreType.DMA((_NBI,)),
        pltpu.SemaphoreType.DMA((_NBO,)),
    ],
)(_sc_body)


def kernel(moe_hidden_states, mlp_hidden_states):
    out_tc = _tc_combine(moe_hidden_states, mlp_hidden_states)
    out_sc = _sc_combine(moe_hidden_states, mlp_hidden_states)
    return lax.dynamic_update_slice(out_tc, out_sc, (_NT_TC, 0))


# speedup vs baseline: 1.0744x; 1.0744x over previous
import functools

import jax
import jax.numpy as jnp
from jax import lax
from jax.experimental import pallas as pl
from jax.experimental.pallas import tpu as pltpu
from jax.experimental.pallas import tpu_sc as plsc

_INV_SQRT2 = 0.7071067811865476

_NT = 16384
_H = 4096

_NT_TC = 11264
_NT_SC = _NT - _NT_TC

_BLOCK_T = 256


def _tc_body(moe_ref, mlp_ref, out_ref):
    out_ref[...] = (moe_ref[:, 0, :] + moe_ref[:, 1, :] + mlp_ref[...]) * _INV_SQRT2


def _tc_combine(moe, mlp):
    return pl.pallas_call(
        _tc_body,
        grid=(_NT_TC // _BLOCK_T,),
        in_specs=[
            pl.BlockSpec((_BLOCK_T, 2, _H), lambda i: (i, 0, 0)),
            pl.BlockSpec((_BLOCK_T, _H), lambda i: (i, 0)),
        ],
        out_specs=pl.BlockSpec((_BLOCK_T, _H), lambda i: (i, 0)),
        out_shape=jax.ShapeDtypeStruct((_NT, _H), jnp.float32),
    )(moe, mlp)


def _stitch_body(full_ref, sc_ref, out_ref):
    del full_ref
    out_ref[...] = sc_ref[...]


def _stitch(full, out_sc):
    return pl.pallas_call(
        _stitch_body,
        grid=(_NT_SC // _BLOCK_T,),
        in_specs=[
            pl.BlockSpec(memory_space=pl.ANY),
            pl.BlockSpec((_BLOCK_T, _H), lambda i: (i, 0)),
        ],
        out_specs=pl.BlockSpec((_BLOCK_T, _H),
                               lambda i: (i + _NT_TC // _BLOCK_T, 0)),
        out_shape=jax.ShapeDtypeStruct((_NT, _H), jnp.float32),
        input_output_aliases={0: 0},
    )(full, out_sc)


_LANES = 16
_NC = 2
_NS = 16
_NW = _NC * _NS
_TOK_PER_W = _NT_SC // _NW
_T = 2
_NCHUNK = _TOK_PER_W // _T
_NBI = 4
_NBO = 2


def _sc_body(moe_hbm, mlp_hbm, out_hbm, moe_v, mlp_v, out_v,
             sem_moe, sem_mlp, sem_out):
    wid = lax.axis_index("s") * _NC + lax.axis_index("c")
    base_in = _NT_TC + wid * _TOK_PER_W
    base_out = wid * _TOK_PER_W

    def in_moe(ci, b):
        tok = base_in + ci * _T
        return pltpu.make_async_copy(
            moe_hbm.at[pl.ds(tok, _T)], moe_v.at[b], sem_moe.at[b])

    def in_mlp(ci, b):
        tok = base_in + ci * _T
        return pltpu.make_async_copy(
            mlp_hbm.at[pl.ds(tok, _T)], mlp_v.at[b], sem_mlp.at[b])

    def out_cp(ci, b):
        tok = base_out + ci * _T
        return pltpu.make_async_copy(
            out_v.at[b], out_hbm.at[pl.ds(tok, _T)], sem_out.at[b])

    for b in range(_NBI):
        in_moe(b, b).start()
        in_mlp(b, b).start()

    @pl.loop(0, _NCHUNK, step=_NBI)
    def _(ci):
        for b in range(_NBI):
            cj = ci + b
            ob = b % _NBO
            in_moe(cj, b).wait()
            in_mlp(cj, b).wait()

            @pl.when(cj >= _NBO)
            def _(cj=cj, ob=ob):
                out_cp(cj - _NBO, ob).wait()

            for t in range(_T):
                @plsc.parallel_loop(0, _H // _LANES, unroll=8)
                def _(j, t=t, b=b, ob=ob):
                    sl = pl.ds(j * _LANES, _LANES)
                    out_v[ob, t, sl] = (
                        moe_v[b, t, 0, sl] + moe_v[b, t, 1, sl]
                        + mlp_v[b, t, sl]) * _INV_SQRT2

            out_cp(cj, ob).start()

            @pl.when(cj + _NBI < _NCHUNK)
            def _(cj=cj, b=b):
                in_moe(cj + _NBI, b).start()
                in_mlp(cj + _NBI, b).start()

    for b in range(_NBO):
        out_cp(_NCHUNK - _NBO + b, b).wait()


_sc_combine = functools.partial(
    pl.kernel,
    out_type=jax.ShapeDtypeStruct((_NT_SC, _H), jnp.float32),
    mesh=plsc.VectorSubcoreMesh(
        core_axis_name="c", subcore_axis_name="s",
        num_cores=_NC, num_subcores=_NS),
    scratch_types=[
        pltpu.VMEM((_NBI, _T, 2, _H), jnp.float32),
        pltpu.VMEM((_NBI, _T, _H), jnp.float32),
        pltpu.VMEM((_NBO, _T, _H), jnp.float32),
        pltpu.SemaphoreType.DMA((_NBI,)),
        pltpu.SemaphoreType.DMA((_NBI,)),
        pltpu.SemaphoreType.DMA((_NBO,)),
    ],
)(_sc_body)


def kernel(moe_hidden_states, mlp_hidden_states):
    out_tc = _tc_combine(moe_hidden_states, mlp_hidden_states)
    out_sc = _sc_combine(moe_hidden_states, mlp_hidden_states)
    return _stitch(out_tc, out_sc)


# speedup vs baseline: 1.1463x; 1.0669x over previous
import functools

import jax
import jax.numpy as jnp
from jax import lax
from jax.experimental import pallas as pl
from jax.experimental.pallas import tpu as pltpu
from jax.experimental.pallas import tpu_sc as plsc

_INV_SQRT2 = 0.7071067811865476

_NT = 16384
_H = 4096
_LANES = 16
_NC = 2
_NS = 16
_NW = _NC * _NS
_TOK_PER_W = _NT // _NW
_T = 2
_NCHUNK = _TOK_PER_W // _T
_NBI = 4
_NBO = 2


def _sc_body(moe_hbm, mlp_hbm, out_hbm, moe_v, mlp_v, out_v,
             sem_moe, sem_mlp, sem_out):
    wid = lax.axis_index("s") * _NC + lax.axis_index("c")
    base = wid * _TOK_PER_W

    def in_moe(ci, b):
        tok = base + ci * _T
        return pltpu.make_async_copy(
            moe_hbm.at[pl.ds(tok, _T)], moe_v.at[b], sem_moe.at[b])

    def in_mlp(ci, b):
        tok = base + ci * _T
        return pltpu.make_async_copy(
            mlp_hbm.at[pl.ds(tok, _T)], mlp_v.at[b], sem_mlp.at[b])

    def out_cp(ci, b):
        tok = base + ci * _T
        return pltpu.make_async_copy(
            out_v.at[b], out_hbm.at[pl.ds(tok, _T)], sem_out.at[b])

    for b in range(_NBI):
        in_moe(b, b).start()
        in_mlp(b, b).start()

    @pl.loop(0, _NCHUNK, step=_NBI)
    def _(ci):
        for b in range(_NBI):
            cj = ci + b
            ob = b % _NBO
            in_moe(cj, b).wait()
            in_mlp(cj, b).wait()

            @pl.when(cj >= _NBO)
            def _(cj=cj, ob=ob):
                out_cp(cj - _NBO, ob).wait()

            for t in range(_T):
                @plsc.parallel_loop(0, _H // _LANES, unroll=8)
                def _(j, t=t, b=b, ob=ob):
                    sl = pl.ds(j * _LANES, _LANES)
                    out_v[ob, t, sl] = (
                        moe_v[b, t, 0, sl] + moe_v[b, t, 1, sl]
                        + mlp_v[b, t, sl]) * _INV_SQRT2

            out_cp(cj, ob).start()

            @pl.when(cj + _NBI < _NCHUNK)
            def _(cj=cj, b=b):
                in_moe(cj + _NBI, b).start()
                in_mlp(cj + _NBI, b).start()

    for b in range(_NBO):
        out_cp(_NCHUNK - _NBO + b, b).wait()


_sc_combine = functools.partial(
    pl.kernel,
    out_type=jax.ShapeDtypeStruct((_NT, _H), jnp.float32),
    mesh=plsc.VectorSubcoreMesh(
        core_axis_name="c", subcore_axis_name="s",
        num_cores=_NC, num_subcores=_NS),
    scratch_types=[
        pltpu.VMEM((_NBI, _T, 2, _H), jnp.float32),
        pltpu.VMEM((_NBI, _T, _H), jnp.float32),
        pltpu.VMEM((_NBO, _T, _H), jnp.float32),
        pltpu.SemaphoreType.DMA((_NBI,)),
        pltpu.SemaphoreType.DMA((_NBI,)),
        pltpu.SemaphoreType.DMA((_NBO,)),
    ],
)(_sc_body)


def kernel(moe_hidden_states, mlp_hidden_states):
    return _sc_combine(moe_hidden_states, mlp_hidden_states)
